# baseline (device time: 46899 ns/iter reference)
import jax
import jax.numpy as jnp
from jax import lax
from jax.experimental import pallas as pl
from jax.experimental.pallas import tpu as pltpu

N_DEV = 4
B, SQ, SKV = 2, 512, 512
HQ_LOC, DH = 8, 64
DM = 768
HALF = DM // 2
DQ_LOC = HQ_LOC * DH
ROWS = B * SQ
CHUNK = ROWS // N_DEV
NH = N_DEV - 1


def kernel(x, Wq, K_ext, V_ext, Wo):
    i = lax.axis_index("i")
    Wq_loc = lax.dynamic_slice(Wq, (0, i * DQ_LOC), (DM, DQ_LOC)) * 0.125
    Wo_loc = lax.dynamic_slice(Wo, (i * DQ_LOC, 0), (DQ_LOC, DM))

    def body(x_ref, wq_ref, k_ref, v_ref, wo_ref, out_ref,
             acc_ref, rs_a, rs_b, snd_a, snd_b, ag_ref, ctx_ref,
             send_sems, recv_sems):
        my = lax.axis_index("i")
        left = lax.rem(my + N_DEV - 1, N_DEV)
        right = lax.rem(my + 1, N_DEV)

        barrier_sem = pltpu.get_barrier_semaphore()
        for nbr in (left, right):
            pl.semaphore_signal(
                barrier_sem, inc=1,
                device_id=(nbr,), device_id_type=pl.DeviceIdType.MESH,
            )
        pl.semaphore_wait(barrier_sem, 2)

        qi = lax.broadcasted_iota(jnp.int32, (SQ, SKV), 0)
        ki = lax.broadcasted_iota(jnp.int32, (SQ, SKV), 1)
        d = qi - ki
        mask = ((d <= 128) & (d >= -128)) | (ki < 32) | (qi < 32)
        bias = jnp.where(mask, 0.0, -1e9).astype(jnp.float32)

        for b in range(B):
            xb = x_ref[b, :, :]
            q = jnp.dot(xb, wq_ref[:, :],
                        preferred_element_type=jnp.float32)
            for h in range(HQ_LOC):
                qh = q[:, h * DH:(h + 1) * DH]
                kh = k_ref[b, :, h, :]
                vh = v_ref[b, :, h, :]
                s = lax.dot_general(
                    qh, kh, (((1,), (1,)), ((), ())),
                    preferred_element_type=jnp.float32)
                w = jnp.exp(s + bias)
                denom = jnp.sum(w, axis=-1, keepdims=True)
                ctx = jnp.dot(w, vh, preferred_element_type=jnp.float32)
                ctx_ref[:, h * DH:(h + 1) * DH] = ctx / denom
            acc_ref[pl.ds(b * SQ, SQ), :] = jnp.dot(
                ctx_ref[:, :], wo_ref[:, :],
                preferred_element_type=jnp.float32)

        for s in range(NH):
            csa = lax.rem(my - s + N_DEV, N_DEV)
            cra = lax.rem(my - s - 1 + N_DEV, N_DEV)
            csb = lax.rem(my + s, N_DEV)
            crb = lax.rem(my + s + 1, N_DEV)
            snd_a[:, :] = acc_ref[pl.ds(csa * CHUNK, CHUNK),
                                  :HALF].astype(jnp.bfloat16)
            snd_b[:, :] = acc_ref[pl.ds(csb * CHUNK, CHUNK),
                                  HALF:].astype(jnp.bfloat16)
            ra = pltpu.make_async_remote_copy(
                src_ref=snd_a, dst_ref=rs_a.at[s],
                send_sem=send_sems.at[2 * s], recv_sem=recv_sems.at[2 * s],
                device_id=(right,), device_id_type=pl.DeviceIdType.MESH,
            )
            rb = pltpu.make_async_remote_copy(
                src_ref=snd_b, dst_ref=rs_b.at[s],
                send_sem=send_sems.at[2 * s + 1],
                recv_sem=recv_sems.at[2 * s + 1],
                device_id=(left,), device_id_type=pl.DeviceIdType.MESH,
            )
            ra.start()
            rb.start()
            ra.wait()
            rb.wait()
            acc_ref[pl.ds(cra * CHUNK, CHUNK), :HALF] = (
                acc_ref[pl.ds(cra * CHUNK, CHUNK), :HALF]
                + rs_a[s, :, :].astype(jnp.float32)
            )
            acc_ref[pl.ds(crb * CHUNK, CHUNK), HALF:] = (
                acc_ref[pl.ds(crb * CHUNK, CHUNK), HALF:]
                + rs_b[s, :, :].astype(jnp.float32)
            )

        owna = lax.rem(my + 1, N_DEV)
        ownb = lax.rem(my + N_DEV - 1, N_DEV)
        ag_ref[pl.ds(owna * CHUNK, CHUNK), :HALF] = acc_ref[
            pl.ds(owna * CHUNK, CHUNK), :HALF].astype(jnp.bfloat16)
        ag_ref[pl.ds(ownb * CHUNK, CHUNK), HALF:] = acc_ref[
            pl.ds(ownb * CHUNK, CHUNK), HALF:].astype(jnp.bfloat16)

        for t in range(NH):
            ca = lax.rem(my + 1 - t + N_DEV, N_DEV)
            cb = lax.rem(my - 1 + t + N_DEV, N_DEV)
            ra = pltpu.make_async_remote_copy(
                src_ref=ag_ref.at[pl.ds(ca * CHUNK, CHUNK), pl.ds(0, HALF)],
                dst_ref=ag_ref.at[pl.ds(ca * CHUNK, CHUNK), pl.ds(0, HALF)],
                send_sem=send_sems.at[2 * NH + 2 * t],
                recv_sem=recv_sems.at[2 * NH + 2 * t],
                device_id=(right,), device_id_type=pl.DeviceIdType.MESH,
            )
            rb = pltpu.make_async_remote_copy(
                src_ref=ag_ref.at[pl.ds(cb * CHUNK, CHUNK), pl.ds(HALF, HALF)],
                dst_ref=ag_ref.at[pl.ds(cb * CHUNK, CHUNK), pl.ds(HALF, HALF)],
                send_sem=send_sems.at[2 * NH + 2 * t + 1],
                recv_sem=recv_sems.at[2 * NH + 2 * t + 1],
                device_id=(left,), device_id_type=pl.DeviceIdType.MESH,
            )
            ra.start()
            rb.start()
            ra.wait()
            rb.wait()

        out_ref[0, :, :] = ag_ref[pl.ds(0, SQ), :].astype(jnp.float32)
        out_ref[1, :, :] = ag_ref[pl.ds(SQ, SQ), :].astype(jnp.float32)

    return pl.pallas_call(
        body,
        out_shape=jax.ShapeDtypeStruct((B, SQ, DM), jnp.float32),
        in_specs=[pl.BlockSpec(memory_space=pltpu.VMEM)] * 5,
        out_specs=pl.BlockSpec(memory_space=pltpu.VMEM),
        scratch_shapes=[
            pltpu.VMEM((ROWS, DM), jnp.float32),
            pltpu.VMEM((NH, CHUNK, HALF), jnp.bfloat16),
            pltpu.VMEM((NH, CHUNK, HALF), jnp.bfloat16),
            pltpu.VMEM((CHUNK, HALF), jnp.bfloat16),
            pltpu.VMEM((CHUNK, HALF), jnp.bfloat16),
            pltpu.VMEM((ROWS, DM), jnp.bfloat16),
            pltpu.VMEM((SQ, DQ_LOC), jnp.float32),
            pltpu.SemaphoreType.DMA((4 * NH,)),
            pltpu.SemaphoreType.DMA((4 * NH,)),
        ],
        compiler_params=pltpu.CompilerParams(collective_id=0),
    )(x, Wq_loc, K_ext, V_ext, Wo_loc)
